# baseline (speedup 1.0000x reference)
import jax
import jax.numpy as jnp
from jax import lax
from jax.experimental import pallas as pl
from jax.experimental.pallas import tpu as pltpu
from jax.experimental.pallas import tpu_sc as plsc

N_NODES = 100000
N_EDGES = 3200000

LANE = 128
NPAD = 100096
SEG = NPAD // 16
ROWS = 25088
ROWS_PER_TILE = ROWS // 32
CHUNK = 16
NCHUNK = ROWS_PER_TILE // CHUNK

_f32 = jnp.float32
_i32 = jnp.int32

_MESH = plsc.VectorSubcoreMesh(core_axis_name="c", subcore_axis_name="s",
                               num_cores=2, num_subcores=16)


def _sc_pass_body(with_cnt, src_hbm, dst_hbm, dst1_hbm, tab_hbm, zseg_hbm,
                  drain_hbm, ones_hbm, agg_out, cnt_out, tab_sp, cnt_sp,
                  srcv, dstv, dstf, vals, acc, onesv, stage, semg, sems):
    cid = lax.axis_index("c")
    sid = lax.axis_index("s")
    seg = sid * SEG

    z16 = jnp.zeros((16,), _f32)

    def zero_acc(i, c):
        acc[pl.ds(pl.multiple_of(i * 16, 16), 16)] = z16
        return c

    lax.fori_loop(0, NPAD // 16, zero_acc, 0)

    if with_cnt:
        pltpu.sync_copy(zseg_hbm, stage)
        pltpu.sync_copy(stage, cnt_sp.at[pl.ds(seg, SEG)])
        pltpu.sync_copy(ones_hbm, onesv)
    pltpu.sync_copy(tab_hbm.at[pl.ds(seg, SEG)], stage)
    pltpu.sync_copy(stage, tab_sp.at[pl.ds(seg, SEG)])
    plsc.subcore_barrier()

    row0 = (cid * 16 + sid) * ROWS_PER_TILE

    def chunk_body(i, carry):
        r = row0 + i * CHUNK
        pltpu.sync_copy(src_hbm.at[pl.ds(r, CHUNK)], srcv)
        pltpu.sync_copy(dst1_hbm.at[pl.ds(r * LANE, CHUNK * LANE)], dstf)
        if with_cnt:
            pltpu.sync_copy(dst_hbm.at[pl.ds(r, CHUNK)], dstv)

        def fire_gather(j, c):
            pltpu.async_copy(tab_sp.at[srcv.at[j]],
                             vals.at[pl.ds(j * LANE, LANE)], semg)
            return c

        lax.fori_loop(0, CHUNK, fire_gather, 0)

        if with_cnt:
            def fire_cnt(j, c):
                pltpu.async_copy(onesv, cnt_sp.at[dstv.at[j]], sems, add=True)
                return c

            lax.fori_loop(0, CHUNK, fire_cnt, 0)

        pltpu.make_async_copy(drain_hbm, vals, semg).wait()

        def acc_vec(m, c):
            off = pl.ds(pl.multiple_of(m * 16, 16), 16)
            plsc.addupdate_scatter(acc, [dstf[off]], vals[off])
            return c

        lax.fori_loop(0, CHUNK * LANE // 16, acc_vec, 0)

        if with_cnt:
            pltpu.make_async_copy(drain_hbm, vals, sems).wait()
        return carry

    lax.fori_loop(0, NCHUNK, chunk_body, 0)

    wid = cid * 16 + sid
    pltpu.sync_copy(acc, agg_out.at[pl.ds(wid * NPAD, NPAD)])
    if with_cnt:
        plsc.subcore_barrier()
        pltpu.sync_copy(cnt_sp.at[pl.ds(seg, SEG)], stage)
        pltpu.sync_copy(stage, cnt_out.at[pl.ds(cid * NPAD + seg, SEG)])


def _make_sc_pass(with_cnt):
    out_type = [jax.ShapeDtypeStruct((32 * NPAD,), _f32)]
    scratch = [
        pltpu.VMEM_SHARED((NPAD,), _f32),
        pltpu.VMEM((CHUNK, LANE), _i32),
        pltpu.VMEM((CHUNK, LANE), _i32),
        pltpu.VMEM((CHUNK * LANE,), _i32),
        pltpu.VMEM((CHUNK * LANE,), _f32),
        pltpu.VMEM((NPAD,), _f32),
        pltpu.VMEM((LANE,), _f32),
        pltpu.VMEM((SEG,), _f32),
        pltpu.SemaphoreType.DMA,
        pltpu.SemaphoreType.DMA,
    ]
    if with_cnt:
        out_type = out_type + [jax.ShapeDtypeStruct((2 * NPAD,), _f32)]
        scratch = scratch[:1] + [pltpu.VMEM_SHARED((NPAD,), _f32)] + scratch[1:]

        def body(src_hbm, dst_hbm, dst1_hbm, tab_hbm, zseg_hbm, drain_hbm,
                 ones_hbm, agg_out, cnt_out, tab_sp, cnt_sp,
                 srcv, dstv, dstf, vals, acc, onesv, stage, semg, sems):
            _sc_pass_body(True, src_hbm, dst_hbm, dst1_hbm, tab_hbm, zseg_hbm,
                          drain_hbm, ones_hbm, agg_out, cnt_out, tab_sp,
                          cnt_sp, srcv, dstv, dstf, vals, acc, onesv, stage,
                          semg, sems)
    else:
        def body(src_hbm, dst_hbm, dst1_hbm, tab_hbm, zseg_hbm, drain_hbm,
                 ones_hbm, agg_out, tab_sp,
                 srcv, dstv, dstf, vals, acc, onesv, stage, semg, sems):
            _sc_pass_body(False, src_hbm, dst_hbm, dst1_hbm, tab_hbm, zseg_hbm,
                          drain_hbm, ones_hbm, agg_out, None, tab_sp, None,
                          srcv, dstv, dstf, vals, acc, onesv, stage,
                          semg, sems)

    return pl.kernel(body, out_type=out_type, mesh=_MESH, scratch_types=scratch,
                     compiler_params=pltpu.CompilerParams(
                         needs_layout_passes=False),
                     name="sage_sc_pass1" if with_cnt else "sage_sc_pass2")


_sc_pass1 = _make_sc_pass(True)
_sc_pass2 = _make_sc_pass(False)


def _mid_body(aggp_ref, cntp_ref, xp_ref, w_ref, s_ref, tpb_ref, degc_ref):
    agg = aggp_ref[0]
    for i in range(1, 32):
        agg = agg + aggp_ref[i]
    deg = cntp_ref[0] + cntp_ref[1]
    degc = jnp.maximum(deg, 1.0)
    a = agg / degc
    xv = xp_ref[...]
    s = jnp.zeros_like(a)
    t = jnp.zeros_like(a)
    for k in range(16):
        h = jnp.maximum(a * w_ref[0, k] + xv * w_ref[1, k] + w_ref[2, k], 0.0)
        s = s + w_ref[3, k] * h
        t = t + w_ref[4, k] * h
    s_ref[...] = s
    tpb_ref[...] = t + w_ref[5, 0]
    degc_ref[...] = degc


_mid_tc = pl.pallas_call(
    _mid_body,
    out_shape=[jax.ShapeDtypeStruct((NPAD // LANE, LANE), _f32)] * 3,
    in_specs=[
        pl.BlockSpec(memory_space=pltpu.VMEM),
        pl.BlockSpec(memory_space=pltpu.VMEM),
        pl.BlockSpec(memory_space=pltpu.VMEM),
        pl.BlockSpec(memory_space=pltpu.SMEM),
    ],
    out_specs=[pl.BlockSpec(memory_space=pltpu.VMEM)] * 3,
    name="sage_tc_mid",
)


def _final_body(aggp_ref, degc_ref, tpb_ref, out_ref):
    agg = aggp_ref[0]
    for i in range(1, 32):
        agg = agg + aggp_ref[i]
    out_ref[...] = agg / degc_ref[...] + tpb_ref[...]


_final_tc = pl.pallas_call(
    _final_body,
    out_shape=jax.ShapeDtypeStruct((NPAD // LANE, LANE), _f32),
    in_specs=[pl.BlockSpec(memory_space=pltpu.VMEM)] * 3,
    out_specs=pl.BlockSpec(memory_space=pltpu.VMEM),
    name="sage_tc_final",
)


def kernel(x, edge_index, W1_l, b1, W1_r, W2_l, b2, W2_r):
    xf = x[:, 0].astype(_f32)
    xpad = jnp.concatenate([xf, jnp.zeros((NPAD - N_NODES,), _f32)])

    src = edge_index[0].astype(_i32)
    dst = edge_index[1].astype(_i32)
    npe = ROWS * LANE - N_EDGES
    pad_ids = lax.iota(_i32, npe)
    src_pad = pad_ids % N_NODES
    dst_pad = N_NODES + pad_ids % (NPAD - N_NODES)
    src2d = jnp.concatenate([src, src_pad]).reshape(ROWS, LANE)
    dst2d = jnp.concatenate([dst, dst_pad]).reshape(ROWS, LANE)

    zseg = jnp.zeros((SEG,), _f32)
    drain = jnp.zeros((CHUNK * LANE,), _f32)
    ones = jnp.ones((LANE,), _f32)
    w = jnp.stack([
        W1_l[:, 0], W1_r[:, 0], b1, W2_l[0, :], W2_r[0, :],
        jnp.full((16,), b2[0], dtype=_f32),
    ]).astype(_f32)

    dst1d = dst2d.reshape(ROWS * LANE)
    agg1p, cntp = _sc_pass1(src2d, dst2d, dst1d, xpad, zseg, drain, ones)
    s, tpb, degc = _mid_tc(
        agg1p.reshape(32, NPAD // LANE, LANE),
        cntp.reshape(2, NPAD // LANE, LANE),
        xpad.reshape(NPAD // LANE, LANE), w)
    (agg2p,) = _sc_pass2(src2d, dst2d, dst1d, s.reshape(NPAD), zseg, drain,
                         ones)
    out = _final_tc(agg2p.reshape(32, NPAD // LANE, LANE), degc, tpb)
    return out.reshape(NPAD)[:N_NODES].reshape(N_NODES, 1)


# speedup vs baseline: 1.8735x; 1.8735x over previous
import jax
import jax.numpy as jnp
from jax import lax
from jax.experimental import pallas as pl
from jax.experimental.pallas import tpu as pltpu
from jax.experimental.pallas import tpu_sc as plsc

N_NODES = 100000
N_EDGES = 3200000

LANE = 128
NPAD = 100096
SEG = NPAD // 16
ROWS = 25088
ROWS_PER_TILE = ROWS // 32
CHUNK = 56
NCHUNK = ROWS_PER_TILE // CHUNK

_f32 = jnp.float32
_i32 = jnp.int32

_MESH = plsc.VectorSubcoreMesh(core_axis_name="c", subcore_axis_name="s",
                               num_cores=2, num_subcores=16)


def _sc_pass_body(with_cnt, src_hbm, dst_hbm, tab_hbm, zseg_hbm, drain_hbm,
                  ones_hbm, agg_out, cnt_out, tab_sp, agg_sp, cnt_sp,
                  srcv_a, dstv_a, vals_a, srcv_b, dstv_b, vals_b,
                  onesv, stage, semi, semg, sems):
    cid = lax.axis_index("c")
    sid = lax.axis_index("s")
    seg = sid * SEG

    pltpu.sync_copy(zseg_hbm, stage)
    pltpu.sync_copy(stage, agg_sp.at[pl.ds(seg, SEG)])
    if with_cnt:
        pltpu.sync_copy(stage, cnt_sp.at[pl.ds(seg, SEG)])
        pltpu.sync_copy(ones_hbm, onesv)
    pltpu.sync_copy(tab_hbm.at[pl.ds(seg, SEG)], stage)
    pltpu.sync_copy(stage, tab_sp.at[pl.ds(seg, SEG)])
    plsc.subcore_barrier()

    row0 = (cid * 16 + sid) * ROWS_PER_TILE
    n_sc_sets = 2 if with_cnt else 1

    def fire_idx(t, sv, dv):
        rr = row0 + jnp.minimum(t, NCHUNK - 1) * CHUNK
        pltpu.async_copy(src_hbm.at[pl.ds(rr, CHUNK)], sv, semi)
        pltpu.async_copy(dst_hbm.at[pl.ds(rr, CHUNK)], dv, semi)

    def wait_idx(sv, dv):
        pltpu.make_async_copy(src_hbm.at[pl.ds(0, CHUNK)], sv, semi).wait()
        pltpu.make_async_copy(src_hbm.at[pl.ds(0, CHUNK)], dv, semi).wait()

    def fire_gathers(sv, vv):
        def g(j, c):
            pltpu.async_copy(tab_sp.at[sv.at[j]], vv.at[j], semg)
            return c

        lax.fori_loop(0, CHUNK, g, 0)

    def drain(sem, n):
        for _ in range(n):
            pltpu.make_async_copy(drain_hbm, vals_a, sem).wait()

    def fire_scatters(vv, dv):
        if with_cnt:
            def gc(j, c):
                pltpu.async_copy(onesv, cnt_sp.at[dv.at[j]], sems, add=True)
                return c

            lax.fori_loop(0, CHUNK, gc, 0)

        def ga(j, c):
            pltpu.async_copy(vv.at[j], agg_sp.at[dv.at[j]], sems, add=True)
            return c

        lax.fori_loop(0, CHUNK, ga, 0)

    def phase(t, this_bufs, next_bufs, drain_prev):
        sv, dv, vv = this_bufs
        svn, dvn, vvn = next_bufs
        if drain_prev is None:
            drain(sems, n_sc_sets)
        else:
            @pl.when(drain_prev)
            def _():
                drain(sems, n_sc_sets)
        fire_idx(t + 1, svn, dvn)
        drain(semg, 1)
        fire_scatters(vv, dv)
        wait_idx(svn, dvn)
        fire_gathers(svn, vvn)

    fire_idx(0, srcv_a, dstv_a)
    wait_idx(srcv_a, dstv_a)
    fire_gathers(srcv_a, vals_a)

    bufs_a = (srcv_a, dstv_a, vals_a)
    bufs_b = (srcv_b, dstv_b, vals_b)

    def pair_body(u, carry):
        t = u * 2
        phase(t, bufs_a, bufs_b, u > 0)
        phase(t + 1, bufs_b, bufs_a, None)
        return carry

    lax.fori_loop(0, NCHUNK // 2, pair_body, 0)

    drain(sems, n_sc_sets)
    drain(semg, 1)

    plsc.subcore_barrier()

    oseg = cid * NPAD + seg
    pltpu.sync_copy(agg_sp.at[pl.ds(seg, SEG)], stage)
    pltpu.sync_copy(stage, agg_out.at[pl.ds(oseg, SEG)])
    if with_cnt:
        pltpu.sync_copy(cnt_sp.at[pl.ds(seg, SEG)], stage)
        pltpu.sync_copy(stage, cnt_out.at[pl.ds(oseg, SEG)])


def _make_sc_pass(with_cnt):
    out_type = [jax.ShapeDtypeStruct((2 * NPAD,), _f32)]
    scratch = [
        pltpu.VMEM_SHARED((NPAD,), _f32),
        pltpu.VMEM_SHARED((NPAD,), _f32),
        pltpu.VMEM((CHUNK, LANE), _i32),
        pltpu.VMEM((CHUNK, LANE), _i32),
        pltpu.VMEM((CHUNK, LANE), _f32),
        pltpu.VMEM((CHUNK, LANE), _i32),
        pltpu.VMEM((CHUNK, LANE), _i32),
        pltpu.VMEM((CHUNK, LANE), _f32),
        pltpu.VMEM((LANE,), _f32),
        pltpu.VMEM((SEG,), _f32),
        pltpu.SemaphoreType.DMA,
        pltpu.SemaphoreType.DMA,
        pltpu.SemaphoreType.DMA,
    ]
    if with_cnt:
        out_type = out_type + [jax.ShapeDtypeStruct((2 * NPAD,), _f32)]
        scratch = scratch[:2] + [pltpu.VMEM_SHARED((NPAD,), _f32)] + scratch[2:]

    if with_cnt:
        def body(src_hbm, dst_hbm, tab_hbm, zseg_hbm, drain_hbm, ones_hbm,
                 agg_out, cnt_out, tab_sp, agg_sp, cnt_sp,
                 srcv_a, dstv_a, vals_a, srcv_b, dstv_b, vals_b,
                 onesv, stage, semi, semg, sems):
            _sc_pass_body(True, src_hbm, dst_hbm, tab_hbm, zseg_hbm, drain_hbm,
                          ones_hbm, agg_out, cnt_out, tab_sp, agg_sp, cnt_sp,
                          srcv_a, dstv_a, vals_a, srcv_b, dstv_b, vals_b,
                          onesv, stage, semi, semg, sems)
    else:
        def body(src_hbm, dst_hbm, tab_hbm, zseg_hbm, drain_hbm, ones_hbm,
                 agg_out, tab_sp, agg_sp,
                 srcv_a, dstv_a, vals_a, srcv_b, dstv_b, vals_b,
                 onesv, stage, semi, semg, sems):
            _sc_pass_body(False, src_hbm, dst_hbm, tab_hbm, zseg_hbm, drain_hbm,
                          ones_hbm, agg_out, None, tab_sp, agg_sp, None,
                          srcv_a, dstv_a, vals_a, srcv_b, dstv_b, vals_b,
                          onesv, stage, semi, semg, sems)

    return pl.kernel(body, out_type=out_type, mesh=_MESH, scratch_types=scratch,
                     compiler_params=pltpu.CompilerParams(
                         needs_layout_passes=False),
                     name="sage_sc_pass1" if with_cnt else "sage_sc_pass2")


_sc_pass1 = _make_sc_pass(True)
_sc_pass2 = _make_sc_pass(False)


def _mid_body(aggp_ref, cntp_ref, xp_ref, w_ref, s_ref, tpb_ref, degc_ref):
    agg = aggp_ref[0] + aggp_ref[1]
    deg = cntp_ref[0] + cntp_ref[1]
    degc = jnp.maximum(deg, 1.0)
    a = agg / degc
    xv = xp_ref[...]
    s = jnp.zeros_like(a)
    t = jnp.zeros_like(a)
    for k in range(16):
        h = jnp.maximum(a * w_ref[0, k] + xv * w_ref[1, k] + w_ref[2, k], 0.0)
        s = s + w_ref[3, k] * h
        t = t + w_ref[4, k] * h
    s_ref[...] = s
    tpb_ref[...] = t + w_ref[5, 0]
    degc_ref[...] = degc


_mid_tc = pl.pallas_call(
    _mid_body,
    out_shape=[jax.ShapeDtypeStruct((NPAD // LANE, LANE), _f32)] * 3,
    in_specs=[
        pl.BlockSpec(memory_space=pltpu.VMEM),
        pl.BlockSpec(memory_space=pltpu.VMEM),
        pl.BlockSpec(memory_space=pltpu.VMEM),
        pl.BlockSpec(memory_space=pltpu.SMEM),
    ],
    out_specs=[pl.BlockSpec(memory_space=pltpu.VMEM)] * 3,
    name="sage_tc_mid",
)


def _final_body(aggp_ref, degc_ref, tpb_ref, out_ref):
    out_ref[...] = (aggp_ref[0] + aggp_ref[1]) / degc_ref[...] + tpb_ref[...]


_final_tc = pl.pallas_call(
    _final_body,
    out_shape=jax.ShapeDtypeStruct((NPAD // LANE, LANE), _f32),
    in_specs=[pl.BlockSpec(memory_space=pltpu.VMEM)] * 3,
    out_specs=pl.BlockSpec(memory_space=pltpu.VMEM),
    name="sage_tc_final",
)


def kernel(x, edge_index, W1_l, b1, W1_r, W2_l, b2, W2_r):
    xf = x[:, 0].astype(_f32)
    xpad = jnp.concatenate([xf, jnp.zeros((NPAD - N_NODES,), _f32)])

    src = edge_index[0].astype(_i32)
    dst = edge_index[1].astype(_i32)
    npe = ROWS * LANE - N_EDGES
    pad_ids = lax.iota(_i32, npe)
    src_pad = pad_ids % N_NODES
    dst_pad = N_NODES + pad_ids % (NPAD - N_NODES)
    src2d = jnp.concatenate([src, src_pad]).reshape(ROWS, LANE)
    dst2d = jnp.concatenate([dst, dst_pad]).reshape(ROWS, LANE)

    zseg = jnp.zeros((SEG,), _f32)
    drain = jnp.zeros((CHUNK, LANE), _f32)
    ones = jnp.ones((LANE,), _f32)
    w = jnp.stack([
        W1_l[:, 0], W1_r[:, 0], b1, W2_l[0, :], W2_r[0, :],
        jnp.full((16,), b2[0], dtype=_f32),
    ]).astype(_f32)

    agg1p, cntp = _sc_pass1(src2d, dst2d, xpad, zseg, drain, ones)
    s, tpb, degc = _mid_tc(
        agg1p.reshape(2, NPAD // LANE, LANE),
        cntp.reshape(2, NPAD // LANE, LANE),
        xpad.reshape(NPAD // LANE, LANE), w)
    (agg2p,) = _sc_pass2(src2d, dst2d, s.reshape(NPAD), zseg, drain, ones)
    out = _final_tc(agg2p.reshape(2, NPAD // LANE, LANE), degc, tpb)
    return out.reshape(NPAD)[:N_NODES].reshape(N_NODES, 1)
